# baseline (speedup 1.0000x reference)
import functools

import jax
import jax.numpy as jnp
from jax import lax
from jax.experimental import pallas as pl
from jax.experimental.pallas import tpu as pltpu
from jax.experimental.pallas import tpu_sc as plsc

BATCH = 16384
EMB_DIM = 64
PAIR_W = 2 * EMB_DIM
NC = 2
NS = 16
NW = NC * NS
B_PER_W = BATCH // NW
GCHUNK = 128
N_GCHUNK = B_PER_W // GCHUNK
L = 16
ROWBLK = 16

TBLK = 8192
TSH = 13
HBLK = TBLK // 2

_DNUMS = lax.GatherDimensionNumbers(
    offset_dims=(), collapsed_slice_dims=(0,), start_index_map=(0,))


def _lane_perm(x, p):
    return lax.gather(x, p[:, None], _DNUMS, slice_sizes=(1,),
                      mode=lax.GatherScatterMode.PROMISE_IN_BOUNDS)


def _transpose_body(t_ref, out_ref):
    x = t_ref[...]
    y = jnp.swapaxes(x, 0, 1)
    out_ref[...] = jnp.concatenate([y[:HBLK, :], y[HBLK:, :]], axis=1)


def _to_pair_rows(table):
    n = table.shape[0]
    t = table.T
    ngrid = (n + TBLK - 1) // TBLK
    return pl.pallas_call(
        _transpose_body,
        grid=(ngrid,),
        in_specs=[pl.BlockSpec((EMB_DIM, TBLK), lambda j: (0, j))],
        out_specs=pl.BlockSpec((HBLK, PAIR_W), lambda j: (j, 0)),
        out_shape=jax.ShapeDtypeStruct((ngrid * HBLK, PAIR_W), jnp.float32),
        compiler_params=pltpu.CompilerParams(
            dimension_semantics=("arbitrary",)),
    )(t)


def _tower_kernel(uid_hbm, iid_hbm, uemb_hbm, iemb_hbm, out_hbm,
                  idx_u, idx_i, pu, pi_, hu_v, hi_v, ru0, ru1, ri0, ri1,
                  out_v, sem0, sem1):
    wid = lax.axis_index("s") * NC + lax.axis_index("c")
    base = wid * B_PER_W

    pltpu.sync_copy(uid_hbm.at[pl.ds(base, B_PER_W)], idx_u)
    pltpu.sync_copy(iid_hbm.at[pl.ds(base, B_PER_W)], idx_i)

    def pair_body(c, _):
        sl = pl.ds(c * L, L)
        for ids_ref, p_ref, h_ref in ((idx_u, pu, hu_v), (idx_i, pi_, hi_v)):
            ids = ids_ref[sl]
            p_ref[sl] = lax.shift_right_logical(ids, 1)
            h_ref[sl] = lax.shift_left(jnp.bitwise_and(ids, 1), 6)
        return _
    lax.fori_loop(0, B_PER_W // L, pair_body, 0)

    ru = (ru0, ru1)
    ri = (ri0, ri1)
    sems = (sem0, sem1)

    def fire(c):
        sl = pl.ds(c * GCHUNK, GCHUNK)
        b = c % 2
        return (pltpu.async_copy(uemb_hbm.at[pu.at[sl]], ru[b], sems[b]),
                pltpu.async_copy(iemb_hbm.at[pi_.at[sl]], ri[b], sems[b]))

    lane = lax.iota(jnp.int32, L)
    perms = [(jnp.arange(L, dtype=jnp.int32) ^ sh) for sh in (8, 4, 2, 1)]

    handles = fire(0)
    for c in range(N_GCHUNK):
        if c + 1 < N_GCHUNK:
            next_handles = fire(c + 1)
        for h in handles:
            h.wait()
        bu_ref, bi_ref = ru[c % 2], ri[c % 2]

        def block_body(blk, _, c=c, bu_ref=bu_ref, bi_ref=bi_ref):
            rsl = pl.ds(c * GCHUNK + blk * ROWBLK, ROWBLK)
            hu = hu_v[rsl]
            hi = hi_v[rsl]
            acc = jnp.zeros((L,), jnp.float32)
            for r in range(ROWBLK):
                row = blk * ROWBLK + r
                off_u = lax.index_in_dim(hu, r, keepdims=False)
                off_i = lax.index_in_dim(hi, r, keepdims=False)
                t = jnp.zeros((L,), jnp.float32)
                for k in range(EMB_DIM // L):
                    u = bu_ref[row, pl.ds(off_u + k * L, L)]
                    v = bi_ref[row, pl.ds(off_i + k * L, L)]
                    t = t + u * v
                for p in perms:
                    t = t + _lane_perm(t, p)
                acc = jnp.where(lane == r, t, acc)
            acc = 1.0 / (1.0 + jnp.exp(-acc))
            out_v[rsl] = acc
            return _

        lax.fori_loop(0, GCHUNK // ROWBLK, block_body, 0)
        if c + 1 < N_GCHUNK:
            handles = next_handles

    pltpu.sync_copy(out_v, out_hbm.at[pl.ds(base, B_PER_W)])


@functools.partial(jax.jit, static_argnames=())
def kernel(user_ids, item_ids, user_emb, item_emb):
    mesh = plsc.VectorSubcoreMesh(core_axis_name="c", subcore_axis_name="s")
    f = pl.kernel(
        _tower_kernel,
        out_type=jax.ShapeDtypeStruct((BATCH,), jnp.float32),
        mesh=mesh,
        compiler_params=pltpu.CompilerParams(use_tc_tiling_on_sc=True),
        scratch_types=[
            pltpu.VMEM((B_PER_W,), jnp.int32),
            pltpu.VMEM((B_PER_W,), jnp.int32),
            pltpu.VMEM((B_PER_W,), jnp.int32),
            pltpu.VMEM((B_PER_W,), jnp.int32),
            pltpu.VMEM((B_PER_W,), jnp.int32),
            pltpu.VMEM((B_PER_W,), jnp.int32),
            pltpu.VMEM((GCHUNK, PAIR_W), jnp.float32),
            pltpu.VMEM((GCHUNK, PAIR_W), jnp.float32),
            pltpu.VMEM((GCHUNK, PAIR_W), jnp.float32),
            pltpu.VMEM((GCHUNK, PAIR_W), jnp.float32),
            pltpu.VMEM((B_PER_W,), jnp.float32),
            pltpu.SemaphoreType.DMA,
            pltpu.SemaphoreType.DMA,
        ],
    )
    n, d = item_emb.shape
    user_pairs = user_emb.reshape(n // 2, 2 * d)
    item_pairs = item_emb.reshape(n // 2, 2 * d)
    return f(user_ids.astype(jnp.int32), item_ids.astype(jnp.int32),
             user_pairs, item_pairs)


# speedup vs baseline: 1.1750x; 1.1750x over previous
import functools

import jax
import jax.numpy as jnp
from jax import lax
from jax.experimental import pallas as pl
from jax.experimental.pallas import tpu as pltpu
from jax.experimental.pallas import tpu_sc as plsc

BATCH = 16384
EMB_DIM = 64
PAIR_W = 2 * EMB_DIM
NC = 2
NS = 16
NW = NC * NS
B_PER_W = BATCH // NW
GCHUNK = 128
N_GCHUNK = B_PER_W // GCHUNK
L = 16
ROWBLK = 16

TBLK = 2048
TSH = 11
HBLK = TBLK // 2

_DNUMS = lax.GatherDimensionNumbers(
    offset_dims=(), collapsed_slice_dims=(0,), start_index_map=(0,))


def _lane_perm(x, p):
    return lax.gather(x, p[:, None], _DNUMS, slice_sizes=(1,),
                      mode=lax.GatherScatterMode.PROMISE_IN_BOUNDS)


def _transpose_body(t_ref, out_ref):
    x = t_ref[...]
    eye = jnp.eye(EMB_DIM, dtype=jnp.float32)
    y = lax.dot_general(x, eye, (((0,), (0,)), ((), ())),
                        preferred_element_type=jnp.float32)
    out_ref[...] = jnp.concatenate([y[:HBLK, :], y[HBLK:, :]], axis=1)


def _to_pair_rows(table):
    n = table.shape[0]
    t = table.T
    ngrid = (n + TBLK - 1) // TBLK
    return pl.pallas_call(
        _transpose_body,
        grid=(ngrid,),
        in_specs=[pl.BlockSpec((EMB_DIM, TBLK), lambda j: (0, j))],
        out_specs=pl.BlockSpec((HBLK, PAIR_W), lambda j: (j, 0)),
        out_shape=jax.ShapeDtypeStruct((ngrid * HBLK, PAIR_W), jnp.float32),
        compiler_params=pltpu.CompilerParams(
            dimension_semantics=("arbitrary",)),
    )(t)


def _tower_kernel(uid_hbm, iid_hbm, uemb_hbm, iemb_hbm, out_hbm,
                  idx_u, idx_i, pu, pi_, hu_v, hi_v, ru0, ru1, ri0, ri1,
                  out_v, sem0, sem1):
    wid = lax.axis_index("s") * NC + lax.axis_index("c")
    base = wid * B_PER_W

    pltpu.sync_copy(uid_hbm.at[pl.ds(base, B_PER_W)], idx_u)
    pltpu.sync_copy(iid_hbm.at[pl.ds(base, B_PER_W)], idx_i)

    def pair_body(c, _):
        sl = pl.ds(c * L, L)
        for ids_ref, p_ref, h_ref in ((idx_u, pu, hu_v), (idx_i, pi_, hi_v)):
            ids = ids_ref[sl]
            g = lax.shift_right_logical(ids, TSH)
            r = jnp.bitwise_and(ids, TBLK - 1)
            ge = lax.shift_right_logical(r, TSH - 1)
            p_ref[sl] = g * HBLK + jnp.bitwise_and(r, HBLK - 1)
            h_ref[sl] = lax.shift_left(ge, 6)
        return _
    lax.fori_loop(0, B_PER_W // L, pair_body, 0)

    ru = (ru0, ru1)
    ri = (ri0, ri1)
    sems = (sem0, sem1)

    def fire(c):
        sl = pl.ds(c * GCHUNK, GCHUNK)
        b = c % 2
        return (pltpu.async_copy(uemb_hbm.at[pu.at[sl]], ru[b], sems[b]),
                pltpu.async_copy(iemb_hbm.at[pi_.at[sl]], ri[b], sems[b]))

    lane = lax.iota(jnp.int32, L)
    perms = [(jnp.arange(L, dtype=jnp.int32) ^ sh) for sh in (8, 4, 2, 1)]

    handles = fire(0)
    for c in range(N_GCHUNK):
        if c + 1 < N_GCHUNK:
            next_handles = fire(c + 1)
        for h in handles:
            h.wait()
        bu_ref, bi_ref = ru[c % 2], ri[c % 2]

        def block_body(blk, _, c=c, bu_ref=bu_ref, bi_ref=bi_ref):
            rsl = pl.ds(c * GCHUNK + blk * ROWBLK, ROWBLK)
            hu = hu_v[rsl]
            hi = hi_v[rsl]
            acc = jnp.zeros((L,), jnp.float32)
            for r in range(ROWBLK):
                row = blk * ROWBLK + r
                off_u = lax.index_in_dim(hu, r, keepdims=False)
                off_i = lax.index_in_dim(hi, r, keepdims=False)
                t = jnp.zeros((L,), jnp.float32)
                for k in range(EMB_DIM // L):
                    u = bu_ref[row, pl.ds(off_u + k * L, L)]
                    v = bi_ref[row, pl.ds(off_i + k * L, L)]
                    t = t + u * v
                for p in perms:
                    t = t + _lane_perm(t, p)
                acc = jnp.where(lane == r, t, acc)
            acc = 1.0 / (1.0 + jnp.exp(-acc))
            out_v[rsl] = acc
            return _

        lax.fori_loop(0, GCHUNK // ROWBLK, block_body, 0)
        if c + 1 < N_GCHUNK:
            handles = next_handles

    pltpu.sync_copy(out_v, out_hbm.at[pl.ds(base, B_PER_W)])


@functools.partial(jax.jit, static_argnames=())
def kernel(user_ids, item_ids, user_emb, item_emb):
    mesh = plsc.VectorSubcoreMesh(core_axis_name="c", subcore_axis_name="s")
    f = pl.kernel(
        _tower_kernel,
        out_type=jax.ShapeDtypeStruct((BATCH,), jnp.float32),
        mesh=mesh,
        compiler_params=pltpu.CompilerParams(use_tc_tiling_on_sc=True),
        scratch_types=[
            pltpu.VMEM((B_PER_W,), jnp.int32),
            pltpu.VMEM((B_PER_W,), jnp.int32),
            pltpu.VMEM((B_PER_W,), jnp.int32),
            pltpu.VMEM((B_PER_W,), jnp.int32),
            pltpu.VMEM((B_PER_W,), jnp.int32),
            pltpu.VMEM((B_PER_W,), jnp.int32),
            pltpu.VMEM((GCHUNK, PAIR_W), jnp.float32),
            pltpu.VMEM((GCHUNK, PAIR_W), jnp.float32),
            pltpu.VMEM((GCHUNK, PAIR_W), jnp.float32),
            pltpu.VMEM((GCHUNK, PAIR_W), jnp.float32),
            pltpu.VMEM((B_PER_W,), jnp.float32),
            pltpu.SemaphoreType.DMA,
            pltpu.SemaphoreType.DMA,
        ],
    )
    return f(user_ids.astype(jnp.int32), item_ids.astype(jnp.int32),
             _to_pair_rows(user_emb), _to_pair_rows(item_emb))


# speedup vs baseline: 2.3294x; 1.9825x over previous
import functools

import jax
import jax.numpy as jnp
from jax import lax
from jax.experimental import pallas as pl
from jax.experimental.pallas import tpu as pltpu
from jax.experimental.pallas import tpu_sc as plsc

BATCH = 16384
EMB_DIM = 64
PAIR_W = 2 * EMB_DIM
NC = 2
NS = 16
NW = NC * NS
B_PER_W = BATCH // NW
GCHUNK = 128
N_GCHUNK = B_PER_W // GCHUNK
L = 16
ROWBLK = 16

TBLK = 16384
TSH = 14
HBLK = TBLK // 2

_DNUMS = lax.GatherDimensionNumbers(
    offset_dims=(), collapsed_slice_dims=(0,), start_index_map=(0,))


def _lane_perm(x, p):
    return lax.gather(x, p[:, None], _DNUMS, slice_sizes=(1,),
                      mode=lax.GatherScatterMode.PROMISE_IN_BOUNDS)


def _transpose_body(t_ref, out_ref):
    x = t_ref[...]
    eye = jnp.eye(EMB_DIM, dtype=jnp.float32)
    y = lax.dot_general(x, eye, (((0,), (0,)), ((), ())),
                        preferred_element_type=jnp.float32)
    out_ref[...] = jnp.concatenate([y[:HBLK, :], y[HBLK:, :]], axis=1)


def _to_pair_rows(table):
    n = table.shape[0]
    t = table.T
    ngrid = (n + TBLK - 1) // TBLK
    return pl.pallas_call(
        _transpose_body,
        grid=(ngrid,),
        in_specs=[pl.BlockSpec((EMB_DIM, TBLK), lambda j: (0, j))],
        out_specs=pl.BlockSpec((HBLK, PAIR_W), lambda j: (j, 0)),
        out_shape=jax.ShapeDtypeStruct((ngrid * HBLK, PAIR_W), jnp.float32),
        compiler_params=pltpu.CompilerParams(
            dimension_semantics=("arbitrary",)),
    )(t)


def _tower_kernel(uid_hbm, iid_hbm, uemb_hbm, iemb_hbm, out_hbm,
                  idx_u, idx_i, pu, pi_, hu_v, hi_v, ru0, ru1, ri0, ri1,
                  out_v, sem0, sem1):
    wid = lax.axis_index("s") * NC + lax.axis_index("c")
    base = wid * B_PER_W

    pltpu.sync_copy(uid_hbm.at[pl.ds(base, B_PER_W)], idx_u)
    pltpu.sync_copy(iid_hbm.at[pl.ds(base, B_PER_W)], idx_i)

    def pair_body(c, _):
        sl = pl.ds(c * L, L)
        for ids_ref, p_ref, h_ref in ((idx_u, pu, hu_v), (idx_i, pi_, hi_v)):
            ids = ids_ref[sl]
            g = lax.shift_right_logical(ids, TSH)
            r = jnp.bitwise_and(ids, TBLK - 1)
            ge = lax.shift_right_logical(r, TSH - 1)
            p_ref[sl] = g * HBLK + jnp.bitwise_and(r, HBLK - 1)
            h_ref[sl] = lax.shift_left(ge, 6)
        return _
    lax.fori_loop(0, B_PER_W // L, pair_body, 0)

    ru = (ru0, ru1)
    ri = (ri0, ri1)
    sems = (sem0, sem1)

    def fire(c):
        sl = pl.ds(c * GCHUNK, GCHUNK)
        b = c % 2
        return (pltpu.async_copy(uemb_hbm.at[pu.at[sl]], ru[b], sems[b]),
                pltpu.async_copy(iemb_hbm.at[pi_.at[sl]], ri[b], sems[b]))

    lane = lax.iota(jnp.int32, L)
    perms = [(jnp.arange(L, dtype=jnp.int32) ^ sh) for sh in (8, 4, 2, 1)]

    handles = fire(0)
    for c in range(N_GCHUNK):
        if c + 1 < N_GCHUNK:
            next_handles = fire(c + 1)
        for h in handles:
            h.wait()
        bu_ref, bi_ref = ru[c % 2], ri[c % 2]

        def block_body(blk, _, c=c, bu_ref=bu_ref, bi_ref=bi_ref):
            rsl = pl.ds(c * GCHUNK + blk * ROWBLK, ROWBLK)
            hu = hu_v[rsl]
            hi = hi_v[rsl]
            acc = jnp.zeros((L,), jnp.float32)
            for r in range(ROWBLK):
                row = blk * ROWBLK + r
                off_u = lax.index_in_dim(hu, r, keepdims=False)
                off_i = lax.index_in_dim(hi, r, keepdims=False)
                t = jnp.zeros((L,), jnp.float32)
                for k in range(EMB_DIM // L):
                    u = bu_ref[row, pl.ds(off_u + k * L, L)]
                    v = bi_ref[row, pl.ds(off_i + k * L, L)]
                    t = t + u * v
                for p in perms:
                    t = t + _lane_perm(t, p)
                acc = jnp.where(lane == r, t, acc)
            acc = 1.0 / (1.0 + jnp.exp(-acc))
            out_v[rsl] = acc
            return _

        lax.fori_loop(0, GCHUNK // ROWBLK, block_body, 0)
        if c + 1 < N_GCHUNK:
            handles = next_handles

    pltpu.sync_copy(out_v, out_hbm.at[pl.ds(base, B_PER_W)])


@functools.partial(jax.jit, static_argnames=())
def kernel(user_ids, item_ids, user_emb, item_emb):
    mesh = plsc.VectorSubcoreMesh(core_axis_name="c", subcore_axis_name="s")
    f = pl.kernel(
        _tower_kernel,
        out_type=jax.ShapeDtypeStruct((BATCH,), jnp.float32),
        mesh=mesh,
        compiler_params=pltpu.CompilerParams(use_tc_tiling_on_sc=True),
        scratch_types=[
            pltpu.VMEM((B_PER_W,), jnp.int32),
            pltpu.VMEM((B_PER_W,), jnp.int32),
            pltpu.VMEM((B_PER_W,), jnp.int32),
            pltpu.VMEM((B_PER_W,), jnp.int32),
            pltpu.VMEM((B_PER_W,), jnp.int32),
            pltpu.VMEM((B_PER_W,), jnp.int32),
            pltpu.VMEM((GCHUNK, PAIR_W), jnp.float32),
            pltpu.VMEM((GCHUNK, PAIR_W), jnp.float32),
            pltpu.VMEM((GCHUNK, PAIR_W), jnp.float32),
            pltpu.VMEM((GCHUNK, PAIR_W), jnp.float32),
            pltpu.VMEM((B_PER_W,), jnp.float32),
            pltpu.SemaphoreType.DMA,
            pltpu.SemaphoreType.DMA,
        ],
    )
    return f(user_ids.astype(jnp.int32), item_ids.astype(jnp.int32),
             _to_pair_rows(user_emb), _to_pair_rows(item_emb))


# speedup vs baseline: 2.4720x; 1.0612x over previous
import functools

import jax
import jax.numpy as jnp
from jax import lax
from jax.experimental import pallas as pl
from jax.experimental.pallas import tpu as pltpu
from jax.experimental.pallas import tpu_sc as plsc

BATCH = 16384
EMB_DIM = 64
PAIR_W = 2 * EMB_DIM
NC = 2
NS = 16
NW = NC * NS
B_PER_W = BATCH // NW
GCHUNK = 128
N_GCHUNK = B_PER_W // GCHUNK
L = 16
ROWBLK = 16

TBLK = 32768
TSH = 15
HBLK = TBLK // 2

_DNUMS = lax.GatherDimensionNumbers(
    offset_dims=(), collapsed_slice_dims=(0,), start_index_map=(0,))


def _lane_perm(x, p):
    return lax.gather(x, p[:, None], _DNUMS, slice_sizes=(1,),
                      mode=lax.GatherScatterMode.PROMISE_IN_BOUNDS)


def _transpose_body(t_ref, out_ref):
    x = t_ref[...]
    eye = jnp.eye(EMB_DIM, dtype=jnp.float32)
    y = lax.dot_general(x, eye, (((0,), (0,)), ((), ())),
                        preferred_element_type=jnp.float32)
    out_ref[...] = jnp.concatenate([y[:HBLK, :], y[HBLK:, :]], axis=1)


def _to_pair_rows(table):
    n = table.shape[0]
    t = table.T
    ngrid = (n + TBLK - 1) // TBLK
    return pl.pallas_call(
        _transpose_body,
        grid=(ngrid,),
        in_specs=[pl.BlockSpec((EMB_DIM, TBLK), lambda j: (0, j))],
        out_specs=pl.BlockSpec((HBLK, PAIR_W), lambda j: (j, 0)),
        out_shape=jax.ShapeDtypeStruct((ngrid * HBLK, PAIR_W), jnp.float32),
        compiler_params=pltpu.CompilerParams(
            dimension_semantics=("arbitrary",)),
    )(t)


def _tower_kernel(uid_hbm, iid_hbm, uemb_hbm, iemb_hbm, out_hbm,
                  idx_u, idx_i, pu, pi_, hu_v, hi_v, ru0, ru1, ri0, ri1,
                  out_v, sem0, sem1):
    wid = lax.axis_index("s") * NC + lax.axis_index("c")
    base = wid * B_PER_W

    pltpu.sync_copy(uid_hbm.at[pl.ds(base, B_PER_W)], idx_u)
    pltpu.sync_copy(iid_hbm.at[pl.ds(base, B_PER_W)], idx_i)

    def pair_body(c, _):
        sl = pl.ds(c * L, L)
        for ids_ref, p_ref, h_ref in ((idx_u, pu, hu_v), (idx_i, pi_, hi_v)):
            ids = ids_ref[sl]
            g = lax.shift_right_logical(ids, TSH)
            r = jnp.bitwise_and(ids, TBLK - 1)
            ge = lax.shift_right_logical(r, TSH - 1)
            p_ref[sl] = g * HBLK + jnp.bitwise_and(r, HBLK - 1)
            h_ref[sl] = lax.shift_left(ge, 6)
        return _
    lax.fori_loop(0, B_PER_W // L, pair_body, 0)

    ru = (ru0, ru1)
    ri = (ri0, ri1)
    sems = (sem0, sem1)

    def fire(c):
        sl = pl.ds(c * GCHUNK, GCHUNK)
        b = c % 2
        return (pltpu.async_copy(uemb_hbm.at[pu.at[sl]], ru[b], sems[b]),
                pltpu.async_copy(iemb_hbm.at[pi_.at[sl]], ri[b], sems[b]))

    lane = lax.iota(jnp.int32, L)
    perms = [(jnp.arange(L, dtype=jnp.int32) ^ sh) for sh in (8, 4, 2, 1)]

    handles = fire(0)
    for c in range(N_GCHUNK):
        if c + 1 < N_GCHUNK:
            next_handles = fire(c + 1)
        for h in handles:
            h.wait()
        bu_ref, bi_ref = ru[c % 2], ri[c % 2]

        def block_body(blk, _, c=c, bu_ref=bu_ref, bi_ref=bi_ref):
            rsl = pl.ds(c * GCHUNK + blk * ROWBLK, ROWBLK)
            hu = hu_v[rsl]
            hi = hi_v[rsl]
            acc = jnp.zeros((L,), jnp.float32)
            for r in range(ROWBLK):
                row = blk * ROWBLK + r
                off_u = lax.index_in_dim(hu, r, keepdims=False)
                off_i = lax.index_in_dim(hi, r, keepdims=False)
                t = jnp.zeros((L,), jnp.float32)
                for k in range(EMB_DIM // L):
                    u = bu_ref[row, pl.ds(off_u + k * L, L)]
                    v = bi_ref[row, pl.ds(off_i + k * L, L)]
                    t = t + u * v
                for p in perms:
                    t = t + _lane_perm(t, p)
                acc = jnp.where(lane == r, t, acc)
            acc = 1.0 / (1.0 + jnp.exp(-acc))
            out_v[rsl] = acc
            return _

        lax.fori_loop(0, GCHUNK // ROWBLK, block_body, 0)
        if c + 1 < N_GCHUNK:
            handles = next_handles

    pltpu.sync_copy(out_v, out_hbm.at[pl.ds(base, B_PER_W)])


@functools.partial(jax.jit, static_argnames=())
def kernel(user_ids, item_ids, user_emb, item_emb):
    mesh = plsc.VectorSubcoreMesh(core_axis_name="c", subcore_axis_name="s")
    f = pl.kernel(
        _tower_kernel,
        out_type=jax.ShapeDtypeStruct((BATCH,), jnp.float32),
        mesh=mesh,
        compiler_params=pltpu.CompilerParams(use_tc_tiling_on_sc=True),
        scratch_types=[
            pltpu.VMEM((B_PER_W,), jnp.int32),
            pltpu.VMEM((B_PER_W,), jnp.int32),
            pltpu.VMEM((B_PER_W,), jnp.int32),
            pltpu.VMEM((B_PER_W,), jnp.int32),
            pltpu.VMEM((B_PER_W,), jnp.int32),
            pltpu.VMEM((B_PER_W,), jnp.int32),
            pltpu.VMEM((GCHUNK, PAIR_W), jnp.float32),
            pltpu.VMEM((GCHUNK, PAIR_W), jnp.float32),
            pltpu.VMEM((GCHUNK, PAIR_W), jnp.float32),
            pltpu.VMEM((GCHUNK, PAIR_W), jnp.float32),
            pltpu.VMEM((B_PER_W,), jnp.float32),
            pltpu.SemaphoreType.DMA,
            pltpu.SemaphoreType.DMA,
        ],
    )
    return f(user_ids.astype(jnp.int32), item_ids.astype(jnp.int32),
             _to_pair_rows(user_emb), _to_pair_rows(item_emb))
